# baseline (speedup 1.0000x reference)
import functools

import jax
import jax.numpy as jnp
from jax import lax
from jax.experimental import pallas as pl
from jax.experimental.pallas import tpu as pltpu
from jax.experimental.pallas import tpu_sc as plsc

N = 10000
E = 320000
D = 128
NC = 2
NS = 16
NW = NC * NS
CH = 96
N1 = 10240
NCHUNK = 105
NGRP = NCHUNK // 3
EPT = NCHUNK * CH
EP = EPT * NW
RPS = N1 // NS

_mesh = plsc.VectorSubcoreMesh(
    core_axis_name="c", subcore_axis_name="s", num_cores=NC, num_subcores=NS
)


@functools.partial(
    pl.kernel,
    out_type=jax.ShapeDtypeStruct((NC, N1), jnp.float32),
    mesh=_mesh,
    scratch_types=[
        pltpu.VMEM((CH,), jnp.int32),
        pltpu.VMEM((CH,), jnp.float32),
        pltpu.VMEM((RPS,), jnp.float32),
        pltpu.VMEM_SHARED((N1,), jnp.float32),
    ],
)
def _deg_kernel(dst_hbm, out_hbm, di_v, ones_v, z_v, dacc_sh):
    c = lax.axis_index("c")
    s = lax.axis_index("s")
    w = s * NC + c

    def initbuf(j, carry):
        ones_v[pl.ds(j * 16, 16)] = jnp.ones((16,), jnp.float32)
        return carry

    lax.fori_loop(0, CH // 16, initbuf, 0)

    def zerobuf(j, carry):
        z_v[pl.ds(j * 16, 16)] = jnp.zeros((16,), jnp.float32)
        return carry

    lax.fori_loop(0, RPS // 16, zerobuf, 0)
    pltpu.sync_copy(z_v, dacc_sh.at[pl.ds(s * RPS, RPS)])
    plsc.subcore_barrier()

    def step(i, carry):
        pltpu.sync_copy(dst_hbm.at[pl.ds(w * EPT + i * CH, CH)], di_v)
        pltpu.sync_copy(ones_v, dacc_sh.at[di_v], add=True)
        return carry

    lax.fori_loop(0, NCHUNK, step, 0)
    plsc.subcore_barrier()
    pltpu.sync_copy(dacc_sh.at[pl.ds(s * RPS, RPS)],
                    out_hbm.at[c].at[pl.ds(s * RPS, RPS)])


@functools.partial(
    pl.kernel,
    out_type=jax.ShapeDtypeStruct((NC, N1, D), jnp.float32),
    mesh=_mesh,
    scratch_types=[
        pltpu.VMEM((CH,), jnp.int32),
        pltpu.VMEM((CH,), jnp.int32),
        pltpu.VMEM((CH,), jnp.int32),
        pltpu.VMEM((CH,), jnp.int32),
        pltpu.VMEM((CH,), jnp.int32),
        pltpu.VMEM((CH,), jnp.int32),
        pltpu.VMEM((CH, D), jnp.float32),
        pltpu.VMEM((CH, D), jnp.float32),
        pltpu.VMEM((CH, D), jnp.float32),
        pltpu.VMEM_SHARED((N1, D), jnp.float32),
        pltpu.SemaphoreType.DMA,
        pltpu.SemaphoreType.DMA,
        pltpu.SemaphoreType.DMA,
    ],
)
def _hop_kernel(x_hbm, src_hbm, dst_hbm, zeros_hbm, out_hbm,
                si0_v, si1_v, si2_v, di0_v, di1_v, di2_v,
                rows0_v, rows1_v, rows2_v, acc_sh, semg0, semg1, semg2):
    c = lax.axis_index("c")
    s = lax.axis_index("s")
    w = s * NC + c

    pltpu.sync_copy(zeros_hbm.at[pl.ds(s * RPS, RPS)],
                    acc_sh.at[pl.ds(s * RPS, RPS)])
    plsc.subcore_barrier()

    si = (si0_v, si1_v, si2_v)
    di = (di0_v, di1_v, di2_v)
    rows = (rows0_v, rows1_v, rows2_v)
    semg = (semg0, semg1, semg2)
    base = w * EPT

    def fetch(chunk_off, slot):
        pltpu.sync_copy(src_hbm.at[pl.ds(chunk_off, CH)], si[slot])
        pltpu.sync_copy(dst_hbm.at[pl.ds(chunk_off, CH)], di[slot])
        pltpu.async_copy(x_hbm.at[si[slot]], rows[slot], semg[slot])

    def drain_scatter(slot):
        pltpu.make_async_copy(
            x_hbm.at[pl.ds(0, CH)], rows[slot], semg[slot]).wait()
        pltpu.sync_copy(rows[slot], acc_sh.at[di[slot]], add=True)

    fetch(base, 0)
    fetch(base + CH, 1)

    def group(g, carry):
        off = base + 3 * g * CH
        fetch(off + 2 * CH, 2)
        drain_scatter(0)
        fetch(off + 3 * CH, 0)
        drain_scatter(1)
        fetch(off + 4 * CH, 1)
        drain_scatter(2)
        return carry

    lax.fori_loop(0, NGRP - 1, group, 0)
    fetch(base + (NCHUNK - 1) * CH, 2)
    drain_scatter(0)
    drain_scatter(1)
    drain_scatter(2)
    plsc.subcore_barrier()
    pltpu.sync_copy(acc_sh.at[pl.ds(s * RPS, RPS)],
                    out_hbm.at[c].at[pl.ds(s * RPS, RPS)])


def _norm_from(d0, d1):
    deg = d0 + d1
    return jnp.where(deg > 0, lax.rsqrt(jnp.maximum(deg, 1e-12)), 0.0)


def _prescale_body(f_ref, d0_ref, d1_ref, o_ref):
    o_ref[...] = f_ref[...] * _norm_from(d0_ref[...], d1_ref[...])


def _mid_body(p0_ref, p1_ref, d0_ref, d1_ref, o_ref):
    nrm = _norm_from(d0_ref[...], d1_ref[...])
    o_ref[...] = (p0_ref[...] + p1_ref[...]) * (nrm * nrm)


def _final_body(p0_ref, p1_ref, d0_ref, d1_ref, w_ref, b_ref, o_ref):
    h = (p0_ref[...] + p1_ref[...]) * _norm_from(d0_ref[...], d1_ref[...])
    o_ref[...] = (
        jnp.dot(h, w_ref[...], preferred_element_type=jnp.float32) + b_ref[...]
    )


_f32 = jnp.float32
_prescale = pl.pallas_call(
    _prescale_body, out_shape=jax.ShapeDtypeStruct((N1, D), _f32))
_mid = pl.pallas_call(
    _mid_body, out_shape=jax.ShapeDtypeStruct((N1, D), _f32))
_final = pl.pallas_call(
    _final_body, out_shape=jax.ShapeDtypeStruct((N1, D), _f32))


def kernel(feat, edge_index, W, b):
    src = edge_index[0].astype(jnp.int32)
    dst = edge_index[1].astype(jnp.int32)
    pad = EP - E
    padidx = N + (jnp.arange(pad, dtype=jnp.int32) % (N1 - N))
    srcp = jnp.concatenate([src, padidx])
    dstp = jnp.concatenate([dst, padidx])
    featp = jnp.concatenate(
        [feat.astype(_f32), jnp.zeros((N1 - N, D), _f32)])
    zeros2d = jnp.zeros((N1, D), _f32)

    degs = _deg_kernel(dstp)
    d0 = degs[0].reshape(N1, 1)
    d1 = degs[1].reshape(N1, 1)

    x0 = _prescale(featp, d0, d1)
    p = _hop_kernel(x0, srcp, dstp, zeros2d)
    x1 = _mid(p[0], p[1], d0, d1)
    q = _hop_kernel(x1, srcp, dstp, zeros2d)
    outp = _final(q[0], q[1], d0, d1, W.astype(_f32),
                  b.astype(_f32).reshape(1, D))
    return outp[:N]


# speedup vs baseline: 1.2311x; 1.2311x over previous
import functools

import jax
import jax.numpy as jnp
from jax import lax
from jax.experimental import pallas as pl
from jax.experimental.pallas import tpu as pltpu
from jax.experimental.pallas import tpu_sc as plsc

N = 10000
E = 320000
D = 128
NC = 2
NS = 16
NW = NC * NS
CH = 128
N1 = 10240
NCHUNK = 80
NPAIR = NCHUNK // 2
EPT = NCHUNK * CH
EP = EPT * NW
TOTCH = EP // CH
RPS = N1 // NS

_mesh = plsc.VectorSubcoreMesh(
    core_axis_name="c", subcore_axis_name="s", num_cores=NC, num_subcores=NS
)


@functools.partial(
    pl.kernel,
    out_type=jax.ShapeDtypeStruct((NC, N1), jnp.float32),
    mesh=_mesh,
    scratch_types=[
        pltpu.VMEM((2, CH), jnp.int32),
        pltpu.VMEM((CH,), jnp.float32),
        pltpu.VMEM((RPS,), jnp.float32),
        pltpu.VMEM_SHARED((N1,), jnp.float32),
    ],
)
def _deg_kernel(eidx_hbm, out_hbm, e_v, ones_v, z_v, dacc_sh):
    c = lax.axis_index("c")
    s = lax.axis_index("s")
    w = s * NC + c

    def initbuf(j, carry):
        ones_v[pl.ds(j * 16, 16)] = jnp.ones((16,), jnp.float32)
        return carry

    lax.fori_loop(0, CH // 16, initbuf, 0)

    def zerobuf(j, carry):
        z_v[pl.ds(j * 16, 16)] = jnp.zeros((16,), jnp.float32)
        return carry

    lax.fori_loop(0, RPS // 16, zerobuf, 0)
    pltpu.sync_copy(z_v, dacc_sh.at[pl.ds(s * RPS, RPS)])
    plsc.subcore_barrier()

    def step(i, carry):
        pltpu.sync_copy(eidx_hbm.at[w * NCHUNK + i], e_v)
        pltpu.sync_copy(ones_v, dacc_sh.at[e_v.at[1]], add=True)
        return carry

    lax.fori_loop(0, NCHUNK, step, 0)
    plsc.subcore_barrier()
    pltpu.sync_copy(dacc_sh.at[pl.ds(s * RPS, RPS)],
                    out_hbm.at[c].at[pl.ds(s * RPS, RPS)])


@functools.partial(
    pl.kernel,
    out_type=jax.ShapeDtypeStruct((NC, N1, D), jnp.float32),
    mesh=_mesh,
    scratch_types=[
        pltpu.VMEM((2, CH), jnp.int32),
        pltpu.VMEM((2, CH), jnp.int32),
        pltpu.VMEM((CH, D), jnp.float32),
        pltpu.VMEM((CH, D), jnp.float32),
        pltpu.VMEM_SHARED((N1, D), jnp.float32),
        pltpu.SemaphoreType.DMA,
        pltpu.SemaphoreType.DMA,
    ],
)
def _hop_kernel(x_hbm, eidx_hbm, zeros_hbm, out_hbm,
                e0_v, e1_v, rows0_v, rows1_v, acc_sh, semg0, semg1):
    c = lax.axis_index("c")
    s = lax.axis_index("s")
    w = s * NC + c

    pltpu.sync_copy(zeros_hbm.at[pl.ds(s * RPS, RPS)],
                    acc_sh.at[pl.ds(s * RPS, RPS)])
    plsc.subcore_barrier()

    eidx = (e0_v, e1_v)
    rows = (rows0_v, rows1_v)
    semg = (semg0, semg1)
    base = w * NCHUNK

    def fetch(chunk, slot):
        pltpu.sync_copy(eidx_hbm.at[chunk], eidx[slot])
        pltpu.async_copy(x_hbm.at[eidx[slot].at[0]], rows[slot], semg[slot])

    def drain_scatter(slot):
        pltpu.make_async_copy(
            x_hbm.at[pl.ds(0, CH)], rows[slot], semg[slot]).wait()
        pltpu.sync_copy(rows[slot], acc_sh.at[eidx[slot].at[1]], add=True)

    fetch(base, 0)

    def pair(i2, carry):
        a = base + 2 * i2
        fetch(a + 1, 1)
        drain_scatter(0)
        fetch(a + 2, 0)
        drain_scatter(1)
        return carry

    lax.fori_loop(0, NPAIR - 1, pair, 0)
    fetch(base + NCHUNK - 1, 1)
    drain_scatter(0)
    drain_scatter(1)
    plsc.subcore_barrier()
    pltpu.sync_copy(acc_sh.at[pl.ds(s * RPS, RPS)],
                    out_hbm.at[c].at[pl.ds(s * RPS, RPS)])


def _norm_from(d0, d1):
    deg = d0 + d1
    return jnp.where(deg > 0, lax.rsqrt(jnp.maximum(deg, 1e-12)), 0.0)


def _prescale_body(f_ref, d0_ref, d1_ref, o_ref):
    o_ref[...] = f_ref[...] * _norm_from(d0_ref[...], d1_ref[...])


def _mid_body(p0_ref, p1_ref, d0_ref, d1_ref, o_ref):
    nrm = _norm_from(d0_ref[...], d1_ref[...])
    o_ref[...] = (p0_ref[...] + p1_ref[...]) * (nrm * nrm)


def _final_body(p0_ref, p1_ref, d0_ref, d1_ref, w_ref, b_ref, o_ref):
    h = (p0_ref[...] + p1_ref[...]) * _norm_from(d0_ref[...], d1_ref[...])
    o_ref[...] = (
        jnp.dot(h, w_ref[...], preferred_element_type=jnp.float32) + b_ref[...]
    )


_f32 = jnp.float32
_prescale = pl.pallas_call(
    _prescale_body, out_shape=jax.ShapeDtypeStruct((N1, D), _f32))
_mid = pl.pallas_call(
    _mid_body, out_shape=jax.ShapeDtypeStruct((N1, D), _f32))
_final = pl.pallas_call(
    _final_body, out_shape=jax.ShapeDtypeStruct((N1, D), _f32))


def kernel(feat, edge_index, W, b):
    src = edge_index[0].astype(jnp.int32)
    dst = edge_index[1].astype(jnp.int32)
    pad = EP - E
    padidx = N + (jnp.arange(pad, dtype=jnp.int32) % (N1 - N))
    srcp = jnp.concatenate([src, padidx]).reshape(TOTCH, CH)
    dstp = jnp.concatenate([dst, padidx]).reshape(TOTCH, CH)
    eidx = jnp.stack([srcp, dstp], axis=1)
    featp = jnp.concatenate(
        [feat.astype(_f32), jnp.zeros((N1 - N, D), _f32)])
    zeros2d = jnp.zeros((N1, D), _f32)

    degs = _deg_kernel(eidx)
    d0 = degs[0].reshape(N1, 1)
    d1 = degs[1].reshape(N1, 1)

    x0 = _prescale(featp, d0, d1)
    p = _hop_kernel(x0, eidx, zeros2d)
    x1 = _mid(p[0], p[1], d0, d1)
    q = _hop_kernel(x1, eidx, zeros2d)
    outp = _final(q[0], q[1], d0, d1, W.astype(_f32),
                  b.astype(_f32).reshape(1, D))
    return outp[:N]


# speedup vs baseline: 1.3347x; 1.0841x over previous
import functools

import jax
import jax.numpy as jnp
from jax import lax
from jax.experimental import pallas as pl
from jax.experimental.pallas import tpu as pltpu
from jax.experimental.pallas import tpu_sc as plsc

N = 10000
E = 320000
D = 128
NC = 2
NS = 16
NW = NC * NS
CH = 128
N1 = 10240
NCHUNK = 80
NPAIR = NCHUNK // 2
EPT = NCHUNK * CH
EP = EPT * NW
TOTCH = EP // CH
RPS = N1 // NS

_mesh = plsc.VectorSubcoreMesh(
    core_axis_name="c", subcore_axis_name="s", num_cores=NC, num_subcores=NS
)


@functools.partial(
    pl.kernel,
    out_type=jax.ShapeDtypeStruct((NC, N1), jnp.float32),
    mesh=_mesh,
    scratch_types=[
        pltpu.VMEM((2, CH), jnp.int32),
        pltpu.VMEM((CH,), jnp.float32),
        pltpu.VMEM((RPS,), jnp.float32),
        pltpu.VMEM_SHARED((N1,), jnp.float32),
    ],
)
def _deg_kernel(eidx_hbm, out_hbm, e_v, ones_v, z_v, dacc_sh):
    c = lax.axis_index("c")
    s = lax.axis_index("s")
    w = s * NC + c

    def initbuf(j, carry):
        ones_v[pl.ds(j * 16, 16)] = jnp.ones((16,), jnp.float32)
        return carry

    lax.fori_loop(0, CH // 16, initbuf, 0)

    def zerobuf(j, carry):
        z_v[pl.ds(j * 16, 16)] = jnp.zeros((16,), jnp.float32)
        return carry

    lax.fori_loop(0, RPS // 16, zerobuf, 0)
    pltpu.sync_copy(z_v, dacc_sh.at[pl.ds(s * RPS, RPS)])
    plsc.subcore_barrier()

    def step(i, carry):
        pltpu.sync_copy(eidx_hbm.at[w * NCHUNK + i], e_v)
        pltpu.sync_copy(ones_v, dacc_sh.at[e_v.at[1]], add=True)
        return carry

    lax.fori_loop(0, NCHUNK, step, 0)
    plsc.subcore_barrier()
    pltpu.sync_copy(dacc_sh.at[pl.ds(s * RPS, RPS)],
                    out_hbm.at[c].at[pl.ds(s * RPS, RPS)])


@functools.partial(
    pl.kernel,
    out_type=jax.ShapeDtypeStruct((NC, N1, D), jnp.float32),
    mesh=_mesh,
    scratch_types=[
        pltpu.VMEM((2, CH), jnp.int32),
        pltpu.VMEM((2, CH), jnp.int32),
        pltpu.VMEM((2, CH), jnp.int32),
        pltpu.VMEM((2, CH), jnp.int32),
        pltpu.VMEM((CH, D), jnp.float32),
        pltpu.VMEM((CH, D), jnp.float32),
        pltpu.VMEM_SHARED((N1, D), jnp.float32),
        pltpu.SemaphoreType.DMA,
        pltpu.SemaphoreType.DMA,
        pltpu.SemaphoreType.DMA,
        pltpu.SemaphoreType.DMA,
        pltpu.SemaphoreType.DMA,
        pltpu.SemaphoreType.DMA,
    ],
)
def _hop_kernel(x_hbm, eidx_hbm, zeros_hbm, out_hbm,
                e0_v, e1_v, e2_v, e3_v, rows0_v, rows1_v, acc_sh,
                semi0, semi1, semi2, semi3, semg0, semg1):
    c = lax.axis_index("c")
    s = lax.axis_index("s")
    w = s * NC + c

    pltpu.sync_copy(zeros_hbm.at[pl.ds(s * RPS, RPS)],
                    acc_sh.at[pl.ds(s * RPS, RPS)])
    plsc.subcore_barrier()

    eidx = (e0_v, e1_v, e2_v, e3_v)
    semi = (semi0, semi1, semi2, semi3)
    rows = (rows0_v, rows1_v)
    semg = (semg0, semg1)
    base = w * NCHUNK

    def fetch_idx(i, j4):
        pltpu.async_copy(eidx_hbm.at[base + i], eidx[j4], semi[j4])

    def gather(j4, j2):
        pltpu.make_async_copy(eidx_hbm.at[0], eidx[j4], semi[j4]).wait()
        pltpu.async_copy(x_hbm.at[eidx[j4].at[0]], rows[j2], semg[j2])

    def scatter(j4, j2):
        pltpu.make_async_copy(
            x_hbm.at[pl.ds(0, CH)], rows[j2], semg[j2]).wait()
        pltpu.sync_copy(rows[j2], acc_sh.at[eidx[j4].at[1]], add=True)

    fetch_idx(0, 0)
    fetch_idx(1, 1)
    fetch_idx(2, 2)
    fetch_idx(3, 3)
    gather(0, 0)

    def group(i4, carry):
        a = 4 * i4
        gather(1, 1)
        scatter(0, 0)
        fetch_idx(a + 4, 0)
        gather(2, 0)
        scatter(1, 1)
        fetch_idx(a + 5, 1)
        gather(3, 1)
        scatter(2, 0)
        fetch_idx(a + 6, 2)
        gather(0, 0)
        scatter(3, 1)
        fetch_idx(a + 7, 3)
        return carry

    lax.fori_loop(0, (NCHUNK - 4) // 4, group, 0)
    gather(1, 1)
    scatter(0, 0)
    gather(2, 0)
    scatter(1, 1)
    gather(3, 1)
    scatter(2, 0)
    scatter(3, 1)
    plsc.subcore_barrier()
    pltpu.sync_copy(acc_sh.at[pl.ds(s * RPS, RPS)],
                    out_hbm.at[c].at[pl.ds(s * RPS, RPS)])


def _norm_from(d0, d1):
    deg = d0 + d1
    return jnp.where(deg > 0, lax.rsqrt(jnp.maximum(deg, 1e-12)), 0.0)


def _prescale_body(f_ref, d0_ref, d1_ref, o_ref):
    o_ref[...] = f_ref[...] * _norm_from(d0_ref[...], d1_ref[...])


def _mid_body(p0_ref, p1_ref, d0_ref, d1_ref, o_ref):
    nrm = _norm_from(d0_ref[...], d1_ref[...])
    o_ref[...] = (p0_ref[...] + p1_ref[...]) * (nrm * nrm)


def _final_body(p0_ref, p1_ref, d0_ref, d1_ref, w_ref, b_ref, o_ref):
    h = (p0_ref[...] + p1_ref[...]) * _norm_from(d0_ref[...], d1_ref[...])
    o_ref[...] = (
        jnp.dot(h, w_ref[...], preferred_element_type=jnp.float32) + b_ref[...]
    )


_f32 = jnp.float32
_prescale = pl.pallas_call(
    _prescale_body, out_shape=jax.ShapeDtypeStruct((N1, D), _f32))
_mid = pl.pallas_call(
    _mid_body, out_shape=jax.ShapeDtypeStruct((N1, D), _f32))
_final = pl.pallas_call(
    _final_body, out_shape=jax.ShapeDtypeStruct((N1, D), _f32))


def kernel(feat, edge_index, W, b):
    src = edge_index[0].astype(jnp.int32)
    dst = edge_index[1].astype(jnp.int32)
    pad = EP - E
    padidx = N + (jnp.arange(pad, dtype=jnp.int32) % (N1 - N))
    srcp = jnp.concatenate([src, padidx]).reshape(TOTCH, CH)
    dstp = jnp.concatenate([dst, padidx]).reshape(TOTCH, CH)
    eidx = jnp.stack([srcp, dstp], axis=1)
    featp = jnp.concatenate(
        [feat.astype(_f32), jnp.zeros((N1 - N, D), _f32)])
    zeros2d = jnp.zeros((N1, D), _f32)

    degs = _deg_kernel(eidx)
    d0 = degs[0].reshape(N1, 1)
    d1 = degs[1].reshape(N1, 1)

    x0 = _prescale(featp, d0, d1)
    p = _hop_kernel(x0, eidx, zeros2d)
    x1 = _mid(p[0], p[1], d0, d1)
    q = _hop_kernel(x1, eidx, zeros2d)
    outp = _final(q[0], q[1], d0, d1, W.astype(_f32),
                  b.astype(_f32).reshape(1, D))
    return outp[:N]


# speedup vs baseline: 1.4309x; 1.0720x over previous
import functools

import jax
import jax.numpy as jnp
from jax import lax
from jax.experimental import pallas as pl
from jax.experimental.pallas import tpu as pltpu
from jax.experimental.pallas import tpu_sc as plsc

N = 10000
E = 320000
D = 128
NC = 2
NS = 16
NW = NC * NS
CH = 128
N1 = 10240
NCHUNK = 80
NPAIR = NCHUNK // 2
EPT = NCHUNK * CH
EP = EPT * NW
TOTCH = EP // CH
RPS = N1 // NS

_mesh = plsc.VectorSubcoreMesh(
    core_axis_name="c", subcore_axis_name="s", num_cores=NC, num_subcores=NS
)


@functools.partial(
    pl.kernel,
    out_type=jax.ShapeDtypeStruct((NC, N1), jnp.float32),
    mesh=_mesh,
    scratch_types=[
        pltpu.VMEM((2, CH), jnp.int32),
        pltpu.VMEM((2, CH), jnp.int32),
        pltpu.VMEM((CH,), jnp.float32),
        pltpu.VMEM((RPS,), jnp.float32),
        pltpu.VMEM_SHARED((N1,), jnp.float32),
        pltpu.SemaphoreType.DMA,
        pltpu.SemaphoreType.DMA,
    ],
)
def _deg_kernel(eidx_hbm, out_hbm, e0_v, e1_v, ones_v, z_v, dacc_sh,
                semi0, semi1):
    c = lax.axis_index("c")
    s = lax.axis_index("s")
    w = s * NC + c
    eidx = (e0_v, e1_v)
    semi = (semi0, semi1)
    base = w * NCHUNK

    def fetch_idx(i, j2):
        pltpu.async_copy(eidx_hbm.at[base + i], eidx[j2], semi[j2])

    def scatter_ones(j2):
        pltpu.make_async_copy(eidx_hbm.at[0], eidx[j2], semi[j2]).wait()
        pltpu.sync_copy(ones_v, dacc_sh.at[eidx[j2].at[1]], add=True)

    fetch_idx(0, 0)
    fetch_idx(1, 1)

    def initbuf(j, carry):
        ones_v[pl.ds(j * 16, 16)] = jnp.ones((16,), jnp.float32)
        return carry

    lax.fori_loop(0, CH // 16, initbuf, 0)

    def zerobuf(j, carry):
        z_v[pl.ds(j * 16, 16)] = jnp.zeros((16,), jnp.float32)
        return carry

    lax.fori_loop(0, RPS // 16, zerobuf, 0)
    pltpu.sync_copy(z_v, dacc_sh.at[pl.ds(s * RPS, RPS)])
    plsc.subcore_barrier()

    def pair(i2, carry):
        a = 2 * i2
        scatter_ones(0)
        fetch_idx(a + 2, 0)
        scatter_ones(1)
        fetch_idx(a + 3, 1)
        return carry

    lax.fori_loop(0, NPAIR - 1, pair, 0)
    scatter_ones(0)
    scatter_ones(1)
    plsc.subcore_barrier()
    pltpu.sync_copy(dacc_sh.at[pl.ds(s * RPS, RPS)],
                    out_hbm.at[c].at[pl.ds(s * RPS, RPS)])


@functools.partial(
    pl.kernel,
    out_type=jax.ShapeDtypeStruct((NC, N1, D), jnp.float32),
    mesh=_mesh,
    scratch_types=[
        pltpu.VMEM((2, CH), jnp.int32),
        pltpu.VMEM((2, CH), jnp.int32),
        pltpu.VMEM((2, CH), jnp.int32),
        pltpu.VMEM((2, CH), jnp.int32),
        pltpu.VMEM((CH, D), jnp.float32),
        pltpu.VMEM((CH, D), jnp.float32),
        pltpu.VMEM_SHARED((N1, D), jnp.float32),
        pltpu.SemaphoreType.DMA,
        pltpu.SemaphoreType.DMA,
        pltpu.SemaphoreType.DMA,
        pltpu.SemaphoreType.DMA,
        pltpu.SemaphoreType.DMA,
        pltpu.SemaphoreType.DMA,
    ],
)
def _hop_kernel(x_hbm, eidx_hbm, zeros_hbm, out_hbm,
                e0_v, e1_v, e2_v, e3_v, rows0_v, rows1_v, acc_sh,
                semi0, semi1, semi2, semi3, semg0, semg1):
    c = lax.axis_index("c")
    s = lax.axis_index("s")
    w = s * NC + c

    pltpu.sync_copy(zeros_hbm.at[pl.ds(s * RPS, RPS)],
                    acc_sh.at[pl.ds(s * RPS, RPS)])
    plsc.subcore_barrier()

    eidx = (e0_v, e1_v, e2_v, e3_v)
    semi = (semi0, semi1, semi2, semi3)
    rows = (rows0_v, rows1_v)
    semg = (semg0, semg1)
    base = w * NCHUNK

    def fetch_idx(i, j4):
        pltpu.async_copy(eidx_hbm.at[base + i], eidx[j4], semi[j4])

    def gather(j4, j2):
        pltpu.make_async_copy(eidx_hbm.at[0], eidx[j4], semi[j4]).wait()
        pltpu.async_copy(x_hbm.at[eidx[j4].at[0]], rows[j2], semg[j2])

    def scatter(j4, j2):
        pltpu.make_async_copy(
            x_hbm.at[pl.ds(0, CH)], rows[j2], semg[j2]).wait()
        pltpu.sync_copy(rows[j2], acc_sh.at[eidx[j4].at[1]], add=True)

    fetch_idx(0, 0)
    fetch_idx(1, 1)
    fetch_idx(2, 2)
    fetch_idx(3, 3)
    gather(0, 0)

    def group(i4, carry):
        a = 4 * i4
        gather(1, 1)
        scatter(0, 0)
        fetch_idx(a + 4, 0)
        gather(2, 0)
        scatter(1, 1)
        fetch_idx(a + 5, 1)
        gather(3, 1)
        scatter(2, 0)
        fetch_idx(a + 6, 2)
        gather(0, 0)
        scatter(3, 1)
        fetch_idx(a + 7, 3)
        return carry

    lax.fori_loop(0, (NCHUNK - 4) // 4, group, 0)
    gather(1, 1)
    scatter(0, 0)
    gather(2, 0)
    scatter(1, 1)
    gather(3, 1)
    scatter(2, 0)
    scatter(3, 1)
    plsc.subcore_barrier()
    pltpu.sync_copy(acc_sh.at[pl.ds(s * RPS, RPS)],
                    out_hbm.at[c].at[pl.ds(s * RPS, RPS)])


def _norm_from(d0, d1):
    deg = d0 + d1
    return jnp.where(deg > 0, lax.rsqrt(jnp.maximum(deg, 1e-12)), 0.0)


def _prescale_body(f_ref, d0_ref, d1_ref, o_ref):
    o_ref[...] = f_ref[...] * _norm_from(d0_ref[...], d1_ref[...])


def _mid_body(p0_ref, p1_ref, d0_ref, d1_ref, o_ref):
    nrm = _norm_from(d0_ref[...], d1_ref[...])
    o_ref[...] = (p0_ref[...] + p1_ref[...]) * (nrm * nrm)


def _final_body(p0_ref, p1_ref, d0_ref, d1_ref, w_ref, b_ref, o_ref):
    h = (p0_ref[...] + p1_ref[...]) * _norm_from(d0_ref[...], d1_ref[...])
    o_ref[...] = (
        jnp.dot(h, w_ref[...], preferred_element_type=jnp.float32) + b_ref[...]
    )


_f32 = jnp.float32
_prescale = pl.pallas_call(
    _prescale_body, out_shape=jax.ShapeDtypeStruct((N1, D), _f32))
_mid = pl.pallas_call(
    _mid_body, out_shape=jax.ShapeDtypeStruct((N1, D), _f32))
_final = pl.pallas_call(
    _final_body, out_shape=jax.ShapeDtypeStruct((N1, D), _f32))


def kernel(feat, edge_index, W, b):
    src = edge_index[0].astype(jnp.int32)
    dst = edge_index[1].astype(jnp.int32)
    pad = EP - E
    padidx = N + (jnp.arange(pad, dtype=jnp.int32) % (N1 - N))
    srcp = jnp.concatenate([src, padidx]).reshape(TOTCH, CH)
    dstp = jnp.concatenate([dst, padidx]).reshape(TOTCH, CH)
    eidx = jnp.stack([srcp, dstp], axis=1)
    featp = jnp.concatenate(
        [feat.astype(_f32), jnp.zeros((N1 - N, D), _f32)])
    zeros2d = jnp.zeros((N1, D), _f32)

    degs = _deg_kernel(eidx)
    d0 = degs[0].reshape(N1, 1)
    d1 = degs[1].reshape(N1, 1)

    x0 = _prescale(featp, d0, d1)
    p = _hop_kernel(x0, eidx, zeros2d)
    x1 = _mid(p[0], p[1], d0, d1)
    q = _hop_kernel(x1, eidx, zeros2d)
    outp = _final(q[0], q[1], d0, d1, W.astype(_f32),
                  b.astype(_f32).reshape(1, D))
    return outp[:N]


# speedup vs baseline: 1.4462x; 1.0107x over previous
import functools

import jax
import jax.numpy as jnp
from jax import lax
from jax.experimental import pallas as pl
from jax.experimental.pallas import tpu as pltpu
from jax.experimental.pallas import tpu_sc as plsc

N = 10000
E = 320000
D = 128
NC = 2
NS = 16
NW = NC * NS
CH = 128
N1 = 10240
NCHUNK = 80
NPAIR = NCHUNK // 2
EPT = NCHUNK * CH
EP = EPT * NW
TOTCH = EP // CH
RPS = N1 // NS

_mesh = plsc.VectorSubcoreMesh(
    core_axis_name="c", subcore_axis_name="s", num_cores=NC, num_subcores=NS
)


@functools.partial(
    pl.kernel,
    out_type=jax.ShapeDtypeStruct((NC, N1), jnp.float32),
    mesh=_mesh,
    scratch_types=[
        pltpu.VMEM((2, CH), jnp.int32),
        pltpu.VMEM((2, CH), jnp.int32),
        pltpu.VMEM((CH,), jnp.float32),
        pltpu.VMEM((RPS,), jnp.float32),
        pltpu.VMEM_SHARED((N1,), jnp.float32),
        pltpu.SemaphoreType.DMA,
        pltpu.SemaphoreType.DMA,
    ],
)
def _deg_kernel(eidx_hbm, out_hbm, e0_v, e1_v, ones_v, z_v, dacc_sh,
                semi0, semi1):
    c = lax.axis_index("c")
    s = lax.axis_index("s")
    w = s * NC + c
    eidx = (e0_v, e1_v)
    semi = (semi0, semi1)
    base = w * NCHUNK

    def fetch_idx(i, j2):
        pltpu.async_copy(eidx_hbm.at[base + i], eidx[j2], semi[j2])

    def scatter_ones(j2):
        pltpu.make_async_copy(eidx_hbm.at[0], eidx[j2], semi[j2]).wait()
        pltpu.sync_copy(ones_v, dacc_sh.at[eidx[j2].at[1]], add=True)

    fetch_idx(0, 0)
    fetch_idx(1, 1)

    def initbuf(j, carry):
        ones_v[pl.ds(j * 16, 16)] = jnp.ones((16,), jnp.float32)
        return carry

    lax.fori_loop(0, CH // 16, initbuf, 0)

    def zerobuf(j, carry):
        z_v[pl.ds(j * 16, 16)] = jnp.zeros((16,), jnp.float32)
        return carry

    lax.fori_loop(0, RPS // 16, zerobuf, 0)
    pltpu.sync_copy(z_v, dacc_sh.at[pl.ds(s * RPS, RPS)])
    plsc.subcore_barrier()

    def pair(i2, carry):
        a = 2 * i2
        scatter_ones(0)
        fetch_idx(a + 2, 0)
        scatter_ones(1)
        fetch_idx(a + 3, 1)
        return carry

    lax.fori_loop(0, NPAIR - 1, pair, 0)
    scatter_ones(0)
    scatter_ones(1)
    plsc.subcore_barrier()
    pltpu.sync_copy(dacc_sh.at[pl.ds(s * RPS, RPS)],
                    out_hbm.at[c].at[pl.ds(s * RPS, RPS)])


@functools.partial(
    pl.kernel,
    out_type=jax.ShapeDtypeStruct((NC, N1, D), jnp.float32),
    mesh=_mesh,
    scratch_types=[
        pltpu.VMEM((2, CH), jnp.int32),
        pltpu.VMEM((2, CH), jnp.int32),
        pltpu.VMEM((2, CH), jnp.int32),
        pltpu.VMEM((2, CH), jnp.int32),
        pltpu.VMEM((CH, D), jnp.float32),
        pltpu.VMEM((CH, D), jnp.float32),
        pltpu.VMEM_SHARED((N1, D), jnp.float32),
        pltpu.SemaphoreType.DMA,
        pltpu.SemaphoreType.DMA,
        pltpu.SemaphoreType.DMA,
        pltpu.SemaphoreType.DMA,
        pltpu.SemaphoreType.DMA,
        pltpu.SemaphoreType.DMA,
    ],
)
def _hop_kernel(x_hbm, eidx_hbm, zeros_hbm, out_hbm,
                e0_v, e1_v, e2_v, e3_v, rows0_v, rows1_v, acc_sh,
                semi0, semi1, semi2, semi3, semg0, semg1):
    c = lax.axis_index("c")
    s = lax.axis_index("s")
    w = s * NC + c

    eidx = (e0_v, e1_v, e2_v, e3_v)
    semi = (semi0, semi1, semi2, semi3)
    rows = (rows0_v, rows1_v)
    semg = (semg0, semg1)
    base = w * NCHUNK

    def fetch_idx(i, j4):
        pltpu.async_copy(eidx_hbm.at[base + i], eidx[j4], semi[j4])

    def gather(j4, j2):
        pltpu.make_async_copy(eidx_hbm.at[0], eidx[j4], semi[j4]).wait()
        pltpu.async_copy(x_hbm.at[eidx[j4].at[0]], rows[j2], semg[j2])

    def scatter(j4, j2):
        pltpu.make_async_copy(
            x_hbm.at[pl.ds(0, CH)], rows[j2], semg[j2]).wait()
        pltpu.sync_copy(rows[j2], acc_sh.at[eidx[j4].at[1]], add=True)

    fetch_idx(0, 0)
    fetch_idx(1, 1)
    fetch_idx(2, 2)
    fetch_idx(3, 3)
    gather(0, 0)
    pltpu.sync_copy(zeros_hbm.at[pl.ds(s * RPS, RPS)],
                    acc_sh.at[pl.ds(s * RPS, RPS)])
    plsc.subcore_barrier()

    def group(i4, carry):
        a = 4 * i4
        gather(1, 1)
        scatter(0, 0)
        fetch_idx(a + 4, 0)
        gather(2, 0)
        scatter(1, 1)
        fetch_idx(a + 5, 1)
        gather(3, 1)
        scatter(2, 0)
        fetch_idx(a + 6, 2)
        gather(0, 0)
        scatter(3, 1)
        fetch_idx(a + 7, 3)
        return carry

    lax.fori_loop(0, (NCHUNK - 4) // 4, group, 0)
    gather(1, 1)
    scatter(0, 0)
    gather(2, 0)
    scatter(1, 1)
    gather(3, 1)
    scatter(2, 0)
    scatter(3, 1)
    plsc.subcore_barrier()
    pltpu.sync_copy(acc_sh.at[pl.ds(s * RPS, RPS)],
                    out_hbm.at[c].at[pl.ds(s * RPS, RPS)])


def _norm_from(d0, d1):
    deg = d0 + d1
    return jnp.where(deg > 0, lax.rsqrt(jnp.maximum(deg, 1e-12)), 0.0)


def _prescale_body(f_ref, d0_ref, d1_ref, o_ref):
    o_ref[...] = f_ref[...] * _norm_from(d0_ref[...], d1_ref[...])


def _mid_body(p0_ref, p1_ref, d0_ref, d1_ref, o_ref):
    nrm = _norm_from(d0_ref[...], d1_ref[...])
    o_ref[...] = (p0_ref[...] + p1_ref[...]) * (nrm * nrm)


def _final_body(p0_ref, p1_ref, d0_ref, d1_ref, w_ref, b_ref, o_ref):
    h = (p0_ref[...] + p1_ref[...]) * _norm_from(d0_ref[...], d1_ref[...])
    o_ref[...] = (
        jnp.dot(h, w_ref[...], preferred_element_type=jnp.float32) + b_ref[...]
    )


_f32 = jnp.float32
_prescale = pl.pallas_call(
    _prescale_body, out_shape=jax.ShapeDtypeStruct((N1, D), _f32))
_mid = pl.pallas_call(
    _mid_body, out_shape=jax.ShapeDtypeStruct((N1, D), _f32))
_final = pl.pallas_call(
    _final_body, out_shape=jax.ShapeDtypeStruct((N1, D), _f32))


def kernel(feat, edge_index, W, b):
    src = edge_index[0].astype(jnp.int32)
    dst = edge_index[1].astype(jnp.int32)
    pad = EP - E
    padidx = N + (jnp.arange(pad, dtype=jnp.int32) % (N1 - N))
    srcp = jnp.concatenate([src, padidx]).reshape(TOTCH, CH)
    dstp = jnp.concatenate([dst, padidx]).reshape(TOTCH, CH)
    eidx = jnp.stack([srcp, dstp], axis=1)
    featp = jnp.concatenate(
        [feat.astype(_f32), jnp.zeros((N1 - N, D), _f32)])
    zeros2d = jnp.zeros((N1, D), _f32)

    degs = _deg_kernel(eidx)
    d0 = degs[0].reshape(N1, 1)
    d1 = degs[1].reshape(N1, 1)

    x0 = _prescale(featp, d0, d1)
    p = _hop_kernel(x0, eidx, zeros2d)
    x1 = _mid(p[0], p[1], d0, d1)
    q = _hop_kernel(x1, eidx, zeros2d)
    outp = _final(q[0], q[1], d0, d1, W.astype(_f32),
                  b.astype(_f32).reshape(1, D))
    return outp[:N]
